# Initial kernel scaffold; baseline (speedup 1.0000x reference)
import jax
import jax.numpy as jnp
from jax.experimental import pallas as pl


def kernel(x, tables):
    raise NotImplementedError("write your pallas kernel here")


# speedup vs baseline: 1.3582x; 1.3582x over previous
import functools

import jax
import jax.numpy as jnp
from jax import lax
from jax.experimental import pallas as pl
from jax.experimental.pallas import tpu as pltpu
from jax.experimental.pallas import tpu_sc as plsc

B = 16384
NCAT = 26
NCONT = 26
NCOLS = 52
V = 100000
D = 32

NC = 2
NS = 16
NW = NC * NS
RW = B // NW
M = 128
NG = RW // M
GROUP = 128
NGRP = (M * NCAT) // GROUP


def _body(x_hbm, tab_hbm, emb_hbm, cont_hbm, x_v, idx_v, rows_v, cvt_v, sem):
    wid = lax.axis_index("s") * NC + lax.axis_index("c")
    iota = lax.iota(jnp.int32, 16)
    pat_a = iota * V
    pat_b = (iota + 10) * V

    for g in range(NG):
        base = wid * RW + g * M

        pltpu.sync_copy(x_hbm.at[pl.ds(base, M)], x_v)

        @pl.loop(0, M)
        def _build(b):
            ca = x_v[b, pl.ds(NCONT, 16)] + pat_a
            cb = x_v[b, pl.ds(NCONT + 10, 16)] + pat_b
            idx_v[pl.ds(b * NCAT, 16)] = ca
            idx_v[pl.ds(b * NCAT + 10, 16)] = cb
            f0 = x_v[b, pl.ds(0, 16)].astype(jnp.float32)
            f1 = x_v[b, pl.ds(10, 16)].astype(jnp.float32)
            cvt_v[b, pl.ds(0, 16)] = f0
            cvt_v[b, pl.ds(10, 16)] = f1

        @pl.loop(0, NGRP)
        def _fire(j):
            pltpu.async_copy(
                tab_hbm.at[idx_v.at[pl.ds(j * GROUP, GROUP)]],
                rows_v.at[pl.ds(j * GROUP, GROUP)],
                sem,
            )

        pltpu.make_async_copy(tab_hbm.at[pl.ds(0, M * NCAT)], rows_v, sem).wait()

        pltpu.sync_copy(rows_v, emb_hbm.at[pl.ds(base * NCAT, M * NCAT)])
        pltpu.sync_copy(cvt_v, cont_hbm.at[pl.ds(base, M)])


@jax.jit
def _embed(x, tab_flat):
    kfn = pl.kernel(
        _body,
        out_type=(
            jax.ShapeDtypeStruct((B * NCAT, D), jnp.float32),
            jax.ShapeDtypeStruct((B, NCONT), jnp.float32),
        ),
        mesh=plsc.VectorSubcoreMesh(core_axis_name="c", subcore_axis_name="s"),
        compiler_params=pltpu.CompilerParams(use_tc_tiling_on_sc=False),
        scratch_types=[
            pltpu.VMEM((M, NCOLS), jnp.int32),
            pltpu.VMEM((M * NCAT,), jnp.int32),
            pltpu.VMEM((M * NCAT, D), jnp.float32),
            pltpu.VMEM((M, NCONT), jnp.float32),
            pltpu.SemaphoreType.DMA,
        ],
    )
    return kfn(x, tab_flat)


def kernel(x, tables):
    emb, cont = _embed(x, tables.reshape(NCAT * V, D))
    return jnp.concatenate([cont, emb.reshape(B, NCAT * D)], axis=1)
